# baseline (speedup 1.0000x reference)
import functools

import jax
import jax.numpy as jnp
from jax import lax
from jax.experimental import pallas as pl
from jax.experimental.pallas import tpu as pltpu
from jax.experimental.pallas import tpu_sc as plsc

CODEBOOK = 2048
HASH = 1000000
D = 64
L = 16
HIST = 50

NC = 2
NS = 16
NW = NC * NS

C = 128
BT_PER_W = 4
NBUF = 3
DT = D // 8


def _mod_ladder(t, ms):
    for m in ms:
        mm = jnp.int32(m)
        t = jnp.where(t >= mm, t - mm, t)
    return t


def _body(c0_hbm, c1_hbm, c2_hbm, table_hbm, out_hbm,
          c0_v, c1_v, c2_v, i2_v, i3_v, r_v, slab_v, bases_v, rot_v,
          sem_c, sem_g1, sem_g23, sem_out):
    num_chunks = HIST * BT_PER_W
    g_end = jnp.int32(num_chunks)

    wid = lax.axis_index("s") * jnp.int32(NC) + lax.axis_index("c")

    def chunk_hb(cc):
        h = lax.shift_right_logical(cc, jnp.int32(2))
        btl = lax.bitwise_and(cc, jnp.int32(3))
        bt = wid * jnp.int32(BT_PER_W) + btl
        return h, bt

    def issue_codes(cc, b):
        b = jnp.int32(b)
        h, bt = chunk_hb(cc)
        sl = pl.ds(pl.multiple_of(bt * jnp.int32(C), C), C)
        pltpu.async_copy(c0_hbm.at[h, sl], c0_v.at[b], sem_c.at[b])
        pltpu.async_copy(c1_hbm.at[h, sl], c1_v.at[b], sem_c.at[b])
        pltpu.async_copy(c2_hbm.at[h, sl], c2_v.at[b], sem_c.at[b])

    def wait_codes(b):
        b = jnp.int32(b)
        z = jnp.int32(0)
        sl0 = pl.ds(z, C)
        pltpu.make_async_copy(c0_hbm.at[z, sl0], c0_v.at[b],
                              sem_c.at[b]).wait()
        pltpu.make_async_copy(c1_hbm.at[z, sl0], c1_v.at[b],
                              sem_c.at[b]).wait()
        pltpu.make_async_copy(c2_hbm.at[z, sl0], c2_v.at[b],
                              sem_c.at[b]).wait()

    def compute_idx(b):
        b = jnp.int32(b)

        def idx_body(i, carry):
            s = pl.ds(i * jnp.int32(L), L)
            a0 = c0_v[b, s]
            a1 = c1_v[b, s]
            a2 = c2_v[b, s]
            t2 = _mod_ladder(a0 * jnp.int32(CODEBOOK) + a1,
                             (4000000, 2000000, 1000000))
            hi = lax.shift_right_logical(t2, jnp.int32(10))
            lo = lax.bitwise_and(t2, jnp.int32(1023))
            u = (hi * jnp.int32(97152) + lo * jnp.int32(2048) + a2)
            t3 = _mod_ladder(u, (64000000, 32000000, 16000000, 8000000,
                                 4000000, 2000000, 1000000))
            i2_v[b, s] = t2
            i3_v[b, s] = t3
            return carry

        lax.fori_loop(jnp.int32(0), jnp.int32(C // L), idx_body, jnp.int32(0))

    def issue_g1(b):
        b = jnp.int32(b)
        pltpu.async_copy(table_hbm.at[c0_v.at[b]], r_v.at[b], sem_g1.at[b])

    def wait_g1(b):
        b = jnp.int32(b)
        pltpu.make_async_copy(table_hbm.at[c0_v.at[b]], r_v.at[b],
                              sem_g1.at[b]).wait()

    def issue_g23(b):
        b = jnp.int32(b)
        pltpu.async_copy(table_hbm.at[i2_v.at[b]], r_v.at[b], sem_g23.at[b],
                         add=True)
        pltpu.async_copy(table_hbm.at[i3_v.at[b]], r_v.at[b], sem_g23.at[b],
                         add=True)

    def wait_g23(b):
        b = jnp.int32(b)
        pltpu.make_async_copy(table_hbm.at[i2_v.at[b]], r_v.at[b],
                              sem_g23.at[b]).wait()
        pltpu.make_async_copy(table_hbm.at[i3_v.at[b]], r_v.at[b],
                              sem_g23.at[b]).wait()

    def init_bases():
        j = lax.iota(jnp.int32, L)
        for k in range(D // L):
            bases_v[pl.ds(jnp.int32(k * L), L)] = j + jnp.int32(k * L)
        for bl8 in range(C // L):
            for t in range(L):
                rot = lax.bitwise_and(j + jnp.int32(t), jnp.int32(L - 1))
                rot_v[pl.ds(jnp.int32((bl8 * L + t) * L), L)] = (
                    rot + jnp.int32(bl8 * L))

    def transpose(b):
        b = jnp.int32(b)
        slab = slab_v.at[b]
        rv = r_v.at[b]

        for d4 in range(D // L):
            cols = bases_v[pl.ds(jnp.int32(d4 * L), L)]

            @plsc.parallel_loop(jnp.int32(0), jnp.int32(C // L),
                                step=jnp.int32(1), unroll=2)
            def _blocks(bl8):
                base = bl8 * jnp.int32(L * L)
                for t in range(L):
                    rows = rot_v[pl.ds(base + jnp.int32(t * L), L)]
                    v = plsc.load_gather(rv, [rows, cols])
                    plsc.store_scatter(slab, [cols, rows], v)

    def issue_out(cc, b):
        b = jnp.int32(b)
        h, bt = chunk_hb(cc)
        for dt in range(DT):
            src = pl.ds(jnp.int32(dt * 8), 8)
            pltpu.async_copy(slab_v.at[b, src],
                             out_hbm.at[h, jnp.int32(dt), bt],
                             sem_out.at[b])

    def wait_out(b):
        b = jnp.int32(b)
        z = jnp.int32(0)
        for dt in range(DT):
            src = pl.ds(jnp.int32(dt * 8), 8)
            pltpu.make_async_copy(slab_v.at[b, src], out_hbm.at[z, z, z],
                                  sem_out.at[b]).wait()

    init_bases()
    for b in range(NBUF):
        issue_codes(jnp.int32(b), b)
    wait_codes(0)
    compute_idx(0)
    issue_g1(0)
    wait_codes(1)
    compute_idx(1)
    issue_g1(1)
    wait_g1(0)
    issue_g23(0)

    def step_body(c, carry):
        b0 = lax.rem(c, jnp.int32(NBUF))

        @pl.when(c + 2 < g_end)
        def _s1():
            for b in range(NBUF):
                @pl.when(b0 == jnp.int32((b + 1) % NBUF))
                def _do():
                    wait_codes(b)
                    compute_idx(b)
                    issue_g1(b)

        @pl.when(c + 1 < g_end)
        def _s2():
            for b in range(NBUF):
                @pl.when(b0 == jnp.int32((b + 2) % NBUF))
                def _do():
                    wait_g1(b)
                    issue_g23(b)

        for b in range(NBUF):
            @pl.when(b0 == jnp.int32(b))
            def _do():
                wait_g23(b)

                @pl.when(c >= jnp.int32(NBUF))
                def _wait_prev_out():
                    wait_out(b)

                transpose(b)

                @pl.when(c + jnp.int32(NBUF) < g_end)
                def _next_codes():
                    issue_codes(c + jnp.int32(NBUF), b)

                issue_out(c, b)

        return carry

    lax.fori_loop(jnp.int32(0), g_end, step_body, jnp.int32(0))

    for b in range(NBUF):
        wait_out(b)


def kernel(codes_0, codes_1, codes_2, embed_table):
    bsz, h = codes_0.shape
    ct0 = codes_0.astype(jnp.int32).T
    ct1 = codes_1.astype(jnp.int32).T
    ct2 = codes_2.astype(jnp.int32).T

    mesh = plsc.VectorSubcoreMesh(core_axis_name="c", subcore_axis_name="s")
    run = functools.partial(
        pl.kernel,
        out_type=jax.ShapeDtypeStruct((h, DT, bsz // C, 8, C),
                                      jnp.float32),
        mesh=mesh,
        compiler_params=pltpu.CompilerParams(use_tc_tiling_on_sc=False,
                                             needs_layout_passes=False),
        scratch_types=[
            pltpu.VMEM((NBUF, C), jnp.int32),
            pltpu.VMEM((NBUF, C), jnp.int32),
            pltpu.VMEM((NBUF, C), jnp.int32),
            pltpu.VMEM((NBUF, C), jnp.int32),
            pltpu.VMEM((NBUF, C), jnp.int32),
            pltpu.VMEM((NBUF, C, D), jnp.float32),
            pltpu.VMEM((NBUF, D, C), jnp.float32),
            pltpu.VMEM((D,), jnp.int32),
            pltpu.VMEM((C * L,), jnp.int32),
            pltpu.SemaphoreType.DMA((NBUF,)),
            pltpu.SemaphoreType.DMA((NBUF,)),
            pltpu.SemaphoreType.DMA((NBUF,)),
            pltpu.SemaphoreType.DMA((NBUF,)),
        ],
    )(_body)
    out5 = run(ct0, ct1, ct2, embed_table)
    return out5.transpose(2, 4, 0, 1, 3).reshape(bsz, h, D)


# speedup vs baseline: 1.6337x; 1.6337x over previous
import functools

import jax
import jax.numpy as jnp
from jax import lax
from jax.experimental import pallas as pl
from jax.experimental.pallas import tpu as pltpu
from jax.experimental.pallas import tpu_sc as plsc

CODEBOOK = 2048
HASH = 1000000
D = 64
L = 16
HIST = 50

NC = 2
NS = 16
NW = NC * NS

C = 128
BT_PER_W = 4
NBUF = 3
DT = D // 8


def _mod_ladder(t, ms):
    for m in ms:
        mm = jnp.int32(m)
        t = jnp.where(t >= mm, t - mm, t)
    return t


def _body(c0_hbm, c1_hbm, c2_hbm, table_hbm, out_hbm,
          c0_v, c1_v, c2_v, i2_v, i3_v, r_v, slab_v, bases_v,
          sem_c, sem_g1, sem_g23, sem_out):
    num_chunks = HIST * BT_PER_W
    g_end = jnp.int32(num_chunks)

    wid = lax.axis_index("s") * jnp.int32(NC) + lax.axis_index("c")

    def chunk_hb(cc):
        h = lax.shift_right_logical(cc, jnp.int32(2))
        btl = lax.bitwise_and(cc, jnp.int32(3))
        bt = wid * jnp.int32(BT_PER_W) + btl
        return h, bt

    def issue_codes(cc, b):
        b = jnp.int32(b)
        h, bt = chunk_hb(cc)
        sl = pl.ds(pl.multiple_of(bt * jnp.int32(C), C), C)
        pltpu.async_copy(c0_hbm.at[h, sl], c0_v.at[b], sem_c.at[b])
        pltpu.async_copy(c1_hbm.at[h, sl], c1_v.at[b], sem_c.at[b])
        pltpu.async_copy(c2_hbm.at[h, sl], c2_v.at[b], sem_c.at[b])

    def wait_codes(b):
        b = jnp.int32(b)
        z = jnp.int32(0)
        sl0 = pl.ds(z, C)
        pltpu.make_async_copy(c0_hbm.at[z, sl0], c0_v.at[b],
                              sem_c.at[b]).wait()
        pltpu.make_async_copy(c1_hbm.at[z, sl0], c1_v.at[b],
                              sem_c.at[b]).wait()
        pltpu.make_async_copy(c2_hbm.at[z, sl0], c2_v.at[b],
                              sem_c.at[b]).wait()

    def compute_idx(b):
        b = jnp.int32(b)

        def idx_body(i, carry):
            s = pl.ds(i * jnp.int32(L), L)
            a0 = c0_v[b, s]
            a1 = c1_v[b, s]
            a2 = c2_v[b, s]
            t2 = _mod_ladder(a0 * jnp.int32(CODEBOOK) + a1,
                             (4000000, 2000000, 1000000))
            hi = lax.shift_right_logical(t2, jnp.int32(10))
            lo = lax.bitwise_and(t2, jnp.int32(1023))
            u = (hi * jnp.int32(97152) + lo * jnp.int32(2048) + a2)
            t3 = _mod_ladder(u, (64000000, 32000000, 16000000, 8000000,
                                 4000000, 2000000, 1000000))
            i2_v[b, s] = t2
            i3_v[b, s] = t3
            return carry

        lax.fori_loop(jnp.int32(0), jnp.int32(C // L), idx_body, jnp.int32(0))

    def issue_g1(b):
        b = jnp.int32(b)
        pltpu.async_copy(table_hbm.at[c0_v.at[b]], r_v.at[b], sem_g1.at[b])

    def wait_g1(b):
        b = jnp.int32(b)
        pltpu.make_async_copy(table_hbm.at[c0_v.at[b]], r_v.at[b],
                              sem_g1.at[b]).wait()

    def issue_g23(b):
        b = jnp.int32(b)
        pltpu.async_copy(table_hbm.at[i2_v.at[b]], r_v.at[b], sem_g23.at[b],
                         add=True)
        pltpu.async_copy(table_hbm.at[i3_v.at[b]], r_v.at[b], sem_g23.at[b],
                         add=True)

    def wait_g23(b):
        b = jnp.int32(b)
        pltpu.make_async_copy(table_hbm.at[i2_v.at[b]], r_v.at[b],
                              sem_g23.at[b]).wait()
        pltpu.make_async_copy(table_hbm.at[i3_v.at[b]], r_v.at[b],
                              sem_g23.at[b]).wait()

    def init_bases():
        j = lax.iota(jnp.int32, L)
        for k in range(D // L):
            bases_v[pl.ds(jnp.int32(k * L), L)] = j + jnp.int32(k * L)

    def transpose(b):
        b = jnp.int32(b)
        slab = slab_v.at[b]
        RU = 4

        dvecs = [bases_v[pl.ds(jnp.int32(k * L), L)] for k in range(D // L)]

        @plsc.parallel_loop(jnp.int32(0), jnp.int32(C), step=jnp.int32(RU),
                            unroll=4)
        def _rows(bl0):
            for r in range(RU):
                bl = bl0 + jnp.int32(r)
                blv = jnp.full((L,), bl, jnp.int32)
                for k in range(D // L):
                    vals = r_v[b, bl, pl.ds(jnp.int32(k * L), L)]
                    plsc.store_scatter(slab, [dvecs[k], blv], vals)

    def issue_out(cc, b):
        b = jnp.int32(b)
        h, bt = chunk_hb(cc)
        w = pl.ds(jnp.int32(0), C)
        for dt in range(DT):
            src = pl.ds(jnp.int32(dt * 8), 8)
            pltpu.async_copy(slab_v.at[b, src, w],
                             out_hbm.at[h, jnp.int32(dt), bt],
                             sem_out.at[b])

    def wait_out(b):
        b = jnp.int32(b)
        z = jnp.int32(0)
        w = pl.ds(jnp.int32(0), C)
        for dt in range(DT):
            src = pl.ds(jnp.int32(dt * 8), 8)
            pltpu.make_async_copy(slab_v.at[b, src, w], out_hbm.at[z, z, z],
                                  sem_out.at[b]).wait()

    init_bases()
    for b in range(NBUF):
        issue_codes(jnp.int32(b), b)
    wait_codes(0)
    compute_idx(0)
    issue_g1(0)
    wait_codes(1)
    compute_idx(1)
    issue_g1(1)
    wait_g1(0)
    issue_g23(0)

    def step_body(c, carry):
        b0 = lax.rem(c, jnp.int32(NBUF))

        @pl.when(c + 2 < g_end)
        def _s1():
            for b in range(NBUF):
                @pl.when(b0 == jnp.int32((b + 1) % NBUF))
                def _do():
                    wait_codes(b)
                    compute_idx(b)
                    issue_g1(b)

        @pl.when(c + 1 < g_end)
        def _s2():
            for b in range(NBUF):
                @pl.when(b0 == jnp.int32((b + 2) % NBUF))
                def _do():
                    wait_g1(b)
                    issue_g23(b)

        for b in range(NBUF):
            @pl.when(b0 == jnp.int32(b))
            def _do():
                wait_g23(b)

                @pl.when(c >= jnp.int32(NBUF))
                def _wait_prev_out():
                    wait_out(b)

                transpose(b)

                @pl.when(c + jnp.int32(NBUF) < g_end)
                def _next_codes():
                    issue_codes(c + jnp.int32(NBUF), b)

                issue_out(c, b)

        return carry

    lax.fori_loop(jnp.int32(0), g_end, step_body, jnp.int32(0))

    for b in range(NBUF):
        wait_out(b)


def kernel(codes_0, codes_1, codes_2, embed_table):
    bsz, h = codes_0.shape
    ct0 = codes_0.astype(jnp.int32).T
    ct1 = codes_1.astype(jnp.int32).T
    ct2 = codes_2.astype(jnp.int32).T

    mesh = plsc.VectorSubcoreMesh(core_axis_name="c", subcore_axis_name="s")
    run = functools.partial(
        pl.kernel,
        out_type=jax.ShapeDtypeStruct((h, DT, bsz // C, 8, C),
                                      jnp.float32),
        mesh=mesh,
        compiler_params=pltpu.CompilerParams(use_tc_tiling_on_sc=False,
                                             needs_layout_passes=False),
        scratch_types=[
            pltpu.VMEM((NBUF, C), jnp.int32),
            pltpu.VMEM((NBUF, C), jnp.int32),
            pltpu.VMEM((NBUF, C), jnp.int32),
            pltpu.VMEM((NBUF, C), jnp.int32),
            pltpu.VMEM((NBUF, C), jnp.int32),
            pltpu.VMEM((NBUF, C, D), jnp.float32),
            pltpu.VMEM((NBUF, D, C + 1), jnp.float32),
            pltpu.VMEM((D,), jnp.int32),
            pltpu.SemaphoreType.DMA((NBUF,)),
            pltpu.SemaphoreType.DMA((NBUF,)),
            pltpu.SemaphoreType.DMA((NBUF,)),
            pltpu.SemaphoreType.DMA((NBUF,)),
        ],
    )(_body)
    out5 = run(ct0, ct1, ct2, embed_table)
    return out5.transpose(2, 4, 0, 1, 3).reshape(bsz, h, D)
